# Initial kernel scaffold; baseline (speedup 1.0000x reference)
import jax
import jax.numpy as jnp
from jax.experimental import pallas as pl


def kernel(feature_data, edge_info, edge_weights, W_in, b_in, W_out, b_out):
    raise NotImplementedError("write your pallas kernel here")


# speedup vs baseline: 4.1246x; 4.1246x over previous
import functools

import jax
import jax.numpy as jnp
from jax import lax
from jax.experimental import pallas as pl
from jax.experimental.pallas import tpu as pltpu
from jax.experimental.pallas import tpu_sc as plsc

N = 10000
D = 128
H = 128
OUT = 128
E = 320000

NC = 2
NS = 16
NW = NC * NS
EPW = E // NW
C = 80
CPW = EPW // C
RB = N // C
LANES = 16


def _sc_msg_partials(src, dst, w, x):
    mesh = plsc.VectorSubcoreMesh(core_axis_name="c", subcore_axis_name="s")

    @functools.partial(
        pl.kernel,
        mesh=mesh,
        out_type=jax.ShapeDtypeStruct((NC, N, D), jnp.float32),
        scratch_types=[
            pltpu.VMEM((C,), jnp.int32),
            pltpu.VMEM((C,), jnp.int32),
            pltpu.VMEM((C,), jnp.float32),
            pltpu.VMEM((C, D), jnp.float32),
            pltpu.VMEM_SHARED((N, D), jnp.float32),
            pltpu.SemaphoreType.DMA,
        ],
    )
    def k(src_hbm, dst_hbm, w_hbm, x_hbm, out_hbm, src_v, dst_v, w_v,
          rows_v, acc, sem):
        cid = lax.axis_index("c")
        sid = lax.axis_index("s")
        wid = sid * NC + cid

        zeros16 = jnp.zeros((LANES,), jnp.float32)
        for e in range(C):
            for kk in range(D // LANES):
                rows_v[e, pl.ds(kk * LANES, LANES)] = zeros16

        nz = (RB + NS - 1) // NS

        def zacc(j, carry):
            ch = sid + j * NS

            @pl.when(ch < RB)
            def _():
                pltpu.sync_copy(rows_v, acc.at[pl.ds(ch * C, C)])

            return carry

        lax.fori_loop(0, nz, zacc, 0)
        plsc.subcore_barrier()

        def chunk(kc, carry):
            base = wid * EPW + kc * C
            pltpu.sync_copy(src_hbm.at[pl.ds(base, C)], src_v)
            pltpu.sync_copy(dst_hbm.at[pl.ds(base, C)], dst_v)
            pltpu.sync_copy(w_hbm.at[pl.ds(base, C)], w_v)
            pltpu.async_copy(x_hbm.at[src_v], rows_v, sem).wait()
            for j in range(C // LANES):
                w16 = w_v[pl.ds(j * LANES, LANES)]
                for l in range(LANES):
                    wl = lax.gather(
                        w16, jnp.full((LANES, 1), l, jnp.int32),
                        lax.GatherDimensionNumbers(
                            offset_dims=(), collapsed_slice_dims=(0,),
                            start_index_map=(0,)),
                        slice_sizes=(1,),
                        mode=lax.GatherScatterMode.PROMISE_IN_BOUNDS)
                    e = j * LANES + l
                    for kk in range(D // LANES):
                        sl = pl.ds(kk * LANES, LANES)
                        rows_v[e, sl] = rows_v[e, sl] * wl
            pltpu.sync_copy(rows_v, acc.at[dst_v], add=True)
            return carry

        lax.fori_loop(0, CPW, chunk, 0)
        plsc.subcore_barrier()

        def wout(j, carry):
            ch = sid + j * NS

            @pl.when(ch < RB)
            def _():
                pltpu.sync_copy(acc.at[pl.ds(ch * C, C)],
                                out_hbm.at[cid, pl.ds(ch * C, C)])

            return carry

        lax.fori_loop(0, nz, wout, 0)

    return k(src, dst, w, x)


def _dense_body(x_ref, m0_ref, m1_ref, win_ref, bin_ref, wout_ref, bout_ref,
                o_ref):
    xb = x_ref[...]
    mb = m0_ref[...] + m1_ref[...]
    win = win_ref[...]
    h = (jnp.dot(xb, win[:, :D].T, preferred_element_type=jnp.float32)
         + jnp.dot(mb, win[:, D:].T, preferred_element_type=jnp.float32)
         + bin_ref[...])
    h = jnp.where(h > 0, h, jnp.exp(jnp.minimum(h, 0.0)) - 1.0)
    o_ref[...] = (jnp.dot(h, wout_ref[...].T,
                          preferred_element_type=jnp.float32) + bout_ref[...])


def _tc_dense(x, m0, m1, W_in, b_in, W_out, b_out):
    BN = 1000
    grid = (N // BN,)
    return pl.pallas_call(
        _dense_body,
        grid=grid,
        in_specs=[
            pl.BlockSpec((BN, D), lambda i: (i, 0)),
            pl.BlockSpec((BN, D), lambda i: (i, 0)),
            pl.BlockSpec((BN, D), lambda i: (i, 0)),
            pl.BlockSpec((H, 2 * D), lambda i: (0, 0)),
            pl.BlockSpec((1, H), lambda i: (0, 0)),
            pl.BlockSpec((OUT, H), lambda i: (0, 0)),
            pl.BlockSpec((1, OUT), lambda i: (0, 0)),
        ],
        out_specs=pl.BlockSpec((BN, OUT), lambda i: (i, 0)),
        out_shape=jax.ShapeDtypeStruct((N, OUT), jnp.float32),
    )(x, m0, m1, W_in, b_in.reshape(1, H), W_out, b_out.reshape(1, OUT))


def kernel(feature_data, edge_info, edge_weights, W_in, b_in, W_out, b_out):
    src = edge_info[0]
    dst = edge_info[1]
    msgp = _sc_msg_partials(src, dst, edge_weights, feature_data)
    return _tc_dense(feature_data, msgp[0], msgp[1], W_in, b_in, W_out, b_out)
